# baseline (speedup 1.0000x reference)
import functools

import jax
import jax.numpy as jnp
from jax import lax
from jax.experimental import pallas as pl
from jax.experimental.pallas import tpu as pltpu
from jax.experimental.pallas import tpu_sc as plsc

VOCAB = 1000000
DIM = 64
SEQ = 200
BATCH = 1024

NC = 2
NS = 16
NW = NC * NS
NLANE = 16
DREG = DIM // NLANE
DPAD = 128

N_TCOL = 7813
FULL_TCOL = 7812
TP_STEPS = 245

SEQ_PER_W = BATCH // NW
CHUNK_SEQ = 8
CHUNK_ROWS = CHUNK_SEQ * SEQ
N_CHUNKS = SEQ_PER_W // CHUNK_SEQ


def _wid():
    return lax.axis_index("s") * NC + lax.axis_index("c")


def _transpose_slab(in_v, out_v, n_pairs):
    lanes = jnp.arange(NLANE, dtype=jnp.int32)

    def pair_body(p, carry):
        i0 = 2 * p
        for half in range(2):
            cols = jnp.full((NLANE,), i0 + half, dtype=jnp.int32)
            for cc in range(DREG):
                rows = lanes + 16 * cc
                v = plsc.load_gather(in_v, [rows, cols])
                out_v[p, pl.ds(half * 64 + cc * 16, NLANE)] = v
        return carry

    lax.fori_loop(0, n_pairs, pair_body, 0)


def _tp_body(tT_hbm, stg_hbm, in_v, out_v, tail_v):
    wid = _wid()

    def blk(k, carry):
        c = wid + k * NW

        @pl.when(c < FULL_TCOL)
        def _full():
            pltpu.sync_copy(tT_hbm.at[:, pl.ds(c * 128, 128)], in_v)
            _transpose_slab(in_v, out_v, 64)
            pltpu.sync_copy(out_v, stg_hbm.at[pl.ds(c * 64, 64), :])

        @pl.when(c == FULL_TCOL)
        def _tail():
            pltpu.sync_copy(tT_hbm.at[:, pl.ds(FULL_TCOL * 128, 64)], tail_v)
            _transpose_slab(tail_v, out_v, 32)
            pltpu.sync_copy(
                out_v.at[pl.ds(0, 32), :],
                stg_hbm.at[pl.ds(FULL_TCOL * 64, 32), :],
            )

        return carry

    lax.fori_loop(0, TP_STEPS, blk, 0)


def _gather_body(x_hbm, table_hbm, pos_hbm, out_hbm, idx_v, rows_v, pos_v, sem):
    wid = _wid()
    pltpu.sync_copy(pos_hbm, pos_v)

    def chunk_body(i, carry):
        base_row = (wid * SEQ_PER_W + i * CHUNK_SEQ) * SEQ
        pltpu.sync_copy(x_hbm.at[pl.ds(base_row, CHUNK_ROWS)], idx_v)
        pltpu.async_copy(table_hbm.at[idx_v], rows_v, sem).wait()

        def l_body(l, carry_l):
            def s_body(s, carry_s):
                r = s * SEQ + l
                for cc in range(DREG):
                    sl = pl.ds(cc * NLANE, NLANE)
                    rows_v[r, sl] = rows_v[r, sl] + pos_v[l, sl]
                return carry_s

            return lax.fori_loop(0, CHUNK_SEQ, s_body, carry_l)

        lax.fori_loop(0, SEQ, l_body, 0)
        pltpu.sync_copy(
            rows_v, out_hbm.at[pl.ds(base_row, CHUNK_ROWS), pl.ds(0, DIM)]
        )
        return carry

    lax.fori_loop(0, N_CHUNKS, chunk_body, 0)


@jax.jit
def kernel(x, table, pos_emb):
    x_flat = x.reshape(-1).astype(jnp.int32)
    mesh = plsc.VectorSubcoreMesh(core_axis_name="c", subcore_axis_name="s")

    transpose_k = functools.partial(
        pl.kernel,
        mesh=mesh,
        compiler_params=pltpu.CompilerParams(
            use_tc_tiling_on_sc=True, needs_layout_passes=False
        ),
        out_type=jax.ShapeDtypeStruct((VOCAB // 2, DPAD), jnp.float32),
        scratch_types=[
            pltpu.VMEM((DIM, 128), jnp.float32),
            pltpu.VMEM((DIM, 128), jnp.float32),
            pltpu.VMEM((DIM, 64), jnp.float32),
        ],
    )(_tp_body)
    staging = transpose_k(table.T)

    gather_k = functools.partial(
        pl.kernel,
        mesh=mesh,
        compiler_params=pltpu.CompilerParams(use_tc_tiling_on_sc=False),
        out_type=jax.ShapeDtypeStruct((BATCH * SEQ, DPAD), jnp.float32),
        scratch_types=[
            pltpu.VMEM((CHUNK_ROWS,), jnp.int32),
            pltpu.VMEM((CHUNK_ROWS, DIM), jnp.float32),
            pltpu.VMEM((SEQ, DIM), jnp.float32),
            pltpu.SemaphoreType.DMA,
        ],
    )(_gather_body)
    out = gather_k(x_flat, staging.reshape(VOCAB, DIM), pos_emb)
    return out[:, :DIM].reshape(BATCH, SEQ, DIM)


# speedup vs baseline: 3.0468x; 3.0468x over previous
import functools

import jax
import jax.numpy as jnp
from jax import lax
from jax.experimental import pallas as pl
from jax.experimental.pallas import tpu as pltpu
from jax.experimental.pallas import tpu_sc as plsc

VOCAB = 1000000
DIM = 64
SEQ = 200
BATCH = 1024

NC = 2
NS = 16
NW = NC * NS
NLANE = 16
DREG = DIM // NLANE
DPAD = 128

N_TCOL = 7813
FULL_TCOL = 7812
TP_STEPS = 245

SEQ_PER_W = BATCH // NW
CHUNK_SEQ = 8
CHUNK_ROWS = CHUNK_SEQ * SEQ
N_CHUNKS = SEQ_PER_W // CHUNK_SEQ


def _wid():
    return lax.axis_index("s") * NC + lax.axis_index("c")


TP_BLK = 2048


def _tp_tc_body(in_ref, out_ref):
    t = in_ref[...]
    tt = jnp.transpose(t, (1, 0))
    t3 = tt.reshape(TP_BLK // 2, 2, DIM)
    out_ref[:, 0:DIM] = t3[:, 0, :]
    out_ref[:, DIM:DPAD] = t3[:, 1, :]


def _gather_body(x_hbm, table_hbm, pos_hbm, out_hbm, idx_v, rows_v, pos_v, sem):
    wid = _wid()
    pltpu.sync_copy(pos_hbm, pos_v)

    def chunk_body(i, carry):
        base_row = (wid * SEQ_PER_W + i * CHUNK_SEQ) * SEQ
        pltpu.sync_copy(x_hbm.at[pl.ds(base_row, CHUNK_ROWS)], idx_v)
        pltpu.async_copy(table_hbm.at[idx_v], rows_v, sem).wait()

        def l_body(l, carry_l):
            def s_body(s, carry_s):
                r = s * SEQ + l
                for cc in range(DREG):
                    sl = pl.ds(cc * NLANE, NLANE)
                    rows_v[r, sl] = rows_v[r, sl] + pos_v[l, sl]
                return carry_s

            return lax.fori_loop(0, CHUNK_SEQ, s_body, carry_l)

        lax.fori_loop(0, SEQ, l_body, 0)
        pltpu.sync_copy(
            rows_v, out_hbm.at[pl.ds(base_row, CHUNK_ROWS), pl.ds(0, DIM)]
        )
        return carry

    lax.fori_loop(0, N_CHUNKS, chunk_body, 0)


@jax.jit
def kernel(x, table, pos_emb):
    x_flat = x.reshape(-1).astype(jnp.int32)
    mesh = plsc.VectorSubcoreMesh(core_axis_name="c", subcore_axis_name="s")

    n_blk = (VOCAB + TP_BLK - 1) // TP_BLK
    staging = pl.pallas_call(
        _tp_tc_body,
        grid=(n_blk,),
        in_specs=[pl.BlockSpec((DIM, TP_BLK), lambda i: (0, i))],
        out_specs=pl.BlockSpec((TP_BLK // 2, DPAD), lambda i: (i, 0)),
        out_shape=jax.ShapeDtypeStruct((VOCAB // 2, DPAD), jnp.float32),
    )(table.T)

    gather_k = functools.partial(
        pl.kernel,
        mesh=mesh,
        compiler_params=pltpu.CompilerParams(use_tc_tiling_on_sc=False),
        out_type=jax.ShapeDtypeStruct((BATCH * SEQ, DPAD), jnp.float32),
        scratch_types=[
            pltpu.VMEM((CHUNK_ROWS,), jnp.int32),
            pltpu.VMEM((CHUNK_ROWS, DIM), jnp.float32),
            pltpu.VMEM((SEQ, DIM), jnp.float32),
            pltpu.SemaphoreType.DMA,
        ],
    )(_gather_body)
    out = gather_k(x_flat, staging.reshape(VOCAB, DIM), pos_emb)
    return out[:, :DIM].reshape(BATCH, SEQ, DIM)
